# Initial kernel scaffold; baseline (speedup 1.0000x reference)
import jax
import jax.numpy as jnp
from jax.experimental import pallas as pl


def kernel(input, nodes, edge_sources, edge_targets, rij, plane_wave, edge_attr, edge_index, Wg, Wm, W2g, W2, W_agat, att):
    raise NotImplementedError("write your pallas kernel here")


# speedup vs baseline: 2.5965x; 2.5965x over previous
import functools
import math

import jax
import jax.numpy as jnp
from jax import lax
from jax.experimental import pallas as pl
from jax.experimental.pallas import tpu as pltpu
from jax.experimental.pallas import tpu_sc as plsc

N = 10000
E = 320000
D = 64
H = 4

NC = 2
NS = 16
NW = NC * NS
PER_W = E // NW
CH = 80
NCHUNK = PER_W // CH
HP = 16

f32 = jnp.float32


def _mesh():
    return plsc.VectorSubcoreMesh(core_axis_name="c", subcore_axis_name="s")


def _wid():
    return lax.axis_index("s") * NC + lax.axis_index("c")


@functools.cache
def _build_sc_gather_pair():
    @functools.partial(
        pl.kernel, mesh=_mesh(),
        compiler_params=pltpu.CompilerParams(use_tc_tiling_on_sc=False),
        out_type=[jax.ShapeDtypeStruct((E, D), f32),
                  jax.ShapeDtypeStruct((E, D), f32)],
        scratch_types=[pltpu.VMEM((CH,), jnp.int32),
                       pltpu.VMEM((CH,), jnp.int32),
                       pltpu.VMEM((CH, D), f32),
                       pltpu.VMEM((CH, D), f32),
                       pltpu.SemaphoreType.DMA,
                       pltpu.SemaphoreType.DMA],
    )
    def gather_pair(inp_hbm, es_hbm, et_hbm, ni_hbm, nj_hbm,
                    esb, etb, nib, njb, sem1, sem2):
        base = _wid() * PER_W

        def body(j, _):
            off = base + j * CH
            pltpu.sync_copy(es_hbm.at[pl.ds(off, CH)], esb)
            pltpu.sync_copy(et_hbm.at[pl.ds(off, CH)], etb)
            pltpu.async_copy(inp_hbm.at[esb], nib, sem1).wait()
            pltpu.async_copy(inp_hbm.at[etb], njb, sem2).wait()
            pltpu.sync_copy(nib, ni_hbm.at[pl.ds(off, CH)])
            pltpu.sync_copy(njb, nj_hbm.at[pl.ds(off, CH)])
            return 0

        lax.fori_loop(0, NCHUNK, body, 0, unroll=False)

    return gather_pair


def _sc_gather_pair(inp, es, et):
    return _build_sc_gather_pair()(inp, es, et)


@functools.cache
def _build_sc_scatter(width):
    @functools.partial(
        pl.kernel, mesh=_mesh(),
        compiler_params=pltpu.CompilerParams(use_tc_tiling_on_sc=False),
        out_type=[jax.ShapeDtypeStruct((N, width), f32),
                  jax.ShapeDtypeStruct((N, width), f32)],
        scratch_types=[pltpu.VMEM((CH,), jnp.int32),
                       pltpu.VMEM((CH, width), f32),
                       pltpu.VMEM_SHARED((N, width), f32)],
    )
    def scat(val_hbm, es_hbm, zero_hbm, p0_hbm, p1_hbm, idxb, vb, acc_sh):
        cid = lax.axis_index("c")
        sid = lax.axis_index("s")

        @pl.when(sid == 0)
        def _():
            pltpu.sync_copy(zero_hbm, acc_sh)

        plsc.subcore_barrier()
        base = _wid() * PER_W

        def body(j, _):
            off = base + j * CH
            pltpu.sync_copy(es_hbm.at[pl.ds(off, CH)], idxb)
            pltpu.sync_copy(val_hbm.at[pl.ds(off, CH)], vb)
            pltpu.sync_copy(vb, acc_sh.at[idxb], add=True)
            return 0

        lax.fori_loop(0, NCHUNK, body, 0, unroll=False)
        plsc.subcore_barrier()

        @pl.when((sid == 0) & (cid == 0))
        def _():
            pltpu.sync_copy(acc_sh, p0_hbm)

        @pl.when((sid == 0) & (cid == 1))
        def _():
            pltpu.sync_copy(acc_sh, p1_hbm)

    return scat


def _sc_scatter_h(val, es, zero):
    return _build_sc_scatter(HP)(val, es, zero)


def _sc_scatter_d(val, es, zero):
    return _build_sc_scatter(D)(val, es, zero)


@functools.cache
def _build_sc_gather_srow():
    @functools.partial(
        pl.kernel, mesh=_mesh(),
        compiler_params=pltpu.CompilerParams(use_tc_tiling_on_sc=False),
        out_type=[jax.ShapeDtypeStruct((E, HP), f32),
                  jax.ShapeDtypeStruct((E, HP), f32)],
        scratch_types=[pltpu.VMEM((CH,), jnp.int32),
                       pltpu.VMEM((CH, HP), f32),
                       pltpu.VMEM((CH, HP), f32),
                       pltpu.SemaphoreType.DMA,
                       pltpu.SemaphoreType.DMA],
    )
    def gather_srow(p0_hbm, p1_hbm, es_hbm, s0_hbm, s1_hbm,
                    idxb, b0, b1, sem1, sem2):
        base = _wid() * PER_W

        def body(j, _):
            off = base + j * CH
            pltpu.sync_copy(es_hbm.at[pl.ds(off, CH)], idxb)
            pltpu.async_copy(p0_hbm.at[idxb], b0, sem1).wait()
            pltpu.async_copy(p1_hbm.at[idxb], b1, sem2).wait()
            pltpu.sync_copy(b0, s0_hbm.at[pl.ds(off, CH)])
            pltpu.sync_copy(b1, s1_hbm.at[pl.ds(off, CH)])
            return 0

        lax.fori_loop(0, NCHUNK, body, 0, unroll=False)

    return gather_srow


def _sc_gather_srow(p0, p1, es):
    return _build_sc_gather_srow()(p0, p1, es)


_INV_SQRT_BN = 1.0 / math.sqrt(1.0 + 1e-05)


def _sigmoid(x):
    return 1.0 / (1.0 + jnp.exp(-x))


def _softplus(x):
    return jnp.maximum(x, 0.0) + jnp.log(1.0 + jnp.exp(-jnp.abs(x)))


def _elu(x):
    return jnp.where(x > 0.0, x, jnp.exp(jnp.minimum(x, 0.0)) - 1.0)


def _edge_body(ni_ref, nj_ref, rij_ref, pw_ref, ea_ref,
               wg1_ref, wg2_ref, wg3_ref, wm1_ref, wm2_ref, wm3_ref,
               w2g_ref, w2_ref, wat_ref, wab_ref, a1_ref, a2_ref,
               e_ref, ap_ref, u_ref):
    ni = ni_ref[...]
    nj = nj_ref[...]
    r = rij_ref[...]
    r = r + (r == 0.0).astype(f32) * 1e-08
    delta = (ni - nj) / r
    pre_g = ni @ wg1_ref[...] + nj @ wg2_ref[...] + delta @ wg3_ref[...]
    pre_m = ni @ wm1_ref[...] + nj @ wm2_ref[...] + delta @ wm3_ref[...]
    gmat = _sigmoid(pre_g) * _elu(pre_m)
    pw = pw_ref[...]
    gate = _sigmoid(pw @ w2g_ref[...])
    z2 = (pw * gate) @ w2_ref[...]
    ea_w = ea_ref[...] @ wab_ref[...]
    oi = _softplus(ni @ wat_ref[...] + ea_w)
    oj = _softplus(nj @ wat_ref[...] + ea_w)
    alpha = oi @ a1_ref[...] + oj @ a2_ref[...]
    alpha = _softplus(_softplus(alpha) * _INV_SQRT_BN)
    e = jnp.exp(alpha)
    e_ref[:, :H] = e
    e_ref[:, H:] = jnp.zeros((e.shape[0], HP - H), f32)
    ap_ref[...] = gmat * z2
    for h in range(H):
        u_ref[:, h * D:(h + 1) * D] = (
            oj[:, h * D:(h + 1) * D] * gmat * ((1.0 / H) * e[:, h:h + 1]))


BE_A = 2560


def _tc_edge(ni, nj, rij2, pw, ea, wg1, wg2, wg3, wm1, wm2, wm3,
             w2g, w2, wat, wab, a1, a2):
    grid = (E // BE_A,)
    row = lambda i: (i, 0)
    full = lambda i: (0, 0)
    return pl.pallas_call(
        _edge_body,
        grid=grid,
        in_specs=[
            pl.BlockSpec((BE_A, D), row),
            pl.BlockSpec((BE_A, D), row),
            pl.BlockSpec((BE_A, 1), row),
            pl.BlockSpec((BE_A, D), row),
            pl.BlockSpec((BE_A, D), row),
            pl.BlockSpec((D, D), full), pl.BlockSpec((D, D), full),
            pl.BlockSpec((D, D), full), pl.BlockSpec((D, D), full),
            pl.BlockSpec((D, D), full), pl.BlockSpec((D, D), full),
            pl.BlockSpec((D, D), full), pl.BlockSpec((D, D), full),
            pl.BlockSpec((D, H * D), full), pl.BlockSpec((D, H * D), full),
            pl.BlockSpec((H * D, H), full), pl.BlockSpec((H * D, H), full),
        ],
        out_specs=[
            pl.BlockSpec((BE_A, HP), row),
            pl.BlockSpec((BE_A, D), row),
            pl.BlockSpec((BE_A, H * D), row),
        ],
        out_shape=[
            jax.ShapeDtypeStruct((E, HP), f32),
            jax.ShapeDtypeStruct((E, D), f32),
            jax.ShapeDtypeStruct((E, H * D), f32),
        ],
    )(ni, nj, rij2, pw, ea, wg1, wg2, wg3, wm1, wm2, wm3,
      w2g, w2, wat, wab, a1, a2)


def _combine_body(u_ref, ap_ref, s0_ref, s1_ref, z_ref):
    rcp = 1.0 / (s0_ref[:, :H] + s1_ref[:, :H] + 1e-16)
    z = ap_ref[...]
    u = u_ref[...]
    for h in range(H):
        z = z + u[:, h * D:(h + 1) * D] * rcp[:, h:h + 1]
    z_ref[...] = z


BE_B = 3200


def _tc_combine(u, ap, s0, s1):
    grid = (E // BE_B,)
    row = lambda i: (i, 0)
    return pl.pallas_call(
        _combine_body,
        grid=grid,
        in_specs=[
            pl.BlockSpec((BE_B, H * D), row),
            pl.BlockSpec((BE_B, D), row),
            pl.BlockSpec((BE_B, HP), row),
            pl.BlockSpec((BE_B, HP), row),
        ],
        out_specs=pl.BlockSpec((BE_B, D), row),
        out_shape=jax.ShapeDtypeStruct((E, D), f32),
    )(u, ap, s0, s1)


def _final_body(inp_ref, q0_ref, q1_ref, out_ref):
    out_ref[...] = inp_ref[...] + q0_ref[...] + q1_ref[...]


BN = 2000


def _tc_final(inp, q0, q1):
    grid = (N // BN,)
    row = lambda i: (i, 0)
    return pl.pallas_call(
        _final_body,
        grid=grid,
        in_specs=[pl.BlockSpec((BN, D), row)] * 3,
        out_specs=pl.BlockSpec((BN, D), row),
        out_shape=jax.ShapeDtypeStruct((N, D), f32),
    )(inp, q0, q1)


def kernel(input, nodes, edge_sources, edge_targets, rij, plane_wave,
           edge_attr, edge_index, Wg, Wm, W2g, W2, W_agat, att):
    es = edge_sources.astype(jnp.int32)
    et = edge_targets.astype(jnp.int32)
    rij2 = rij.reshape(E, 1)

    att2 = att[0]
    eye = jnp.eye(H, dtype=f32)
    a1 = jnp.einsum("hd,hg->hdg", att2[:, :D], eye).reshape(H * D, H)
    a2 = jnp.einsum("hd,hg->hdg", att2[:, D:], eye).reshape(H * D, H)

    wg1, wg2, wg3 = Wg[:D], Wg[D:2 * D], Wg[2 * D:]
    wm1, wm2, wm3 = Wm[:D], Wm[D:2 * D], Wm[2 * D:]
    wat, wab = W_agat[:D], W_agat[D:]

    ni, nj = _sc_gather_pair(input, es, et)
    e, ap, u = _tc_edge(ni, nj, rij2, plane_wave, edge_attr,
                        wg1, wg2, wg3, wm1, wm2, wm3, W2g, W2, wat, wab,
                        a1, a2)
    zero_h = jnp.zeros((N, HP), dtype=f32)
    p0, p1 = _sc_scatter_h(e, es, zero_h)
    s0, s1 = _sc_gather_srow(p0, p1, es)
    z = _tc_combine(u, ap, s0, s1)
    zero_d = jnp.zeros((N, D), dtype=f32)
    q0, q1 = _sc_scatter_d(z, es, zero_d)
    return _tc_final(input, q0, q1)


# speedup vs baseline: 2.9778x; 1.1469x over previous
import functools
import math

import jax
import jax.numpy as jnp
from jax import lax
from jax.experimental import pallas as pl
from jax.experimental.pallas import tpu as pltpu
from jax.experimental.pallas import tpu_sc as plsc

N = 10000
E = 320000
D = 64
H = 4

NC = 2
NS = 16
NW = NC * NS
PER_W = E // NW
CH = 80
NCHUNK = PER_W // CH
HP = 16

f32 = jnp.float32


def _mesh():
    return plsc.VectorSubcoreMesh(core_axis_name="c", subcore_axis_name="s")


def _wid():
    return lax.axis_index("s") * NC + lax.axis_index("c")


@functools.cache
def _build_sc_gather_pair():
    @functools.partial(
        pl.kernel, mesh=_mesh(),
        compiler_params=pltpu.CompilerParams(use_tc_tiling_on_sc=False),
        out_type=[jax.ShapeDtypeStruct((E, D), f32),
                  jax.ShapeDtypeStruct((E, D), f32)],
        scratch_types=[pltpu.VMEM((CH,), jnp.int32),
                       pltpu.VMEM((CH,), jnp.int32),
                       pltpu.VMEM((CH, D), f32),
                       pltpu.VMEM((CH, D), f32),
                       pltpu.VMEM_SHARED((N, D), f32),
                       pltpu.SemaphoreType.DMA,
                       pltpu.SemaphoreType.DMA,
                       pltpu.SemaphoreType.DMA,
                       pltpu.SemaphoreType.DMA],
    )
    def gather_pair(inp_hbm, es_hbm, et_hbm, ni_hbm, nj_hbm,
                    esb, etb, nib, njb, inp_sh, sem1, sem2, sem3, sem4):
        sid = lax.axis_index("s")

        @pl.when(sid == 0)
        def _():
            pltpu.sync_copy(inp_hbm, inp_sh)

        plsc.subcore_barrier()
        base = _wid() * PER_W

        def body(j, _):
            off = base + j * CH
            c1 = pltpu.async_copy(es_hbm.at[pl.ds(off, CH)], esb, sem1)
            c2 = pltpu.async_copy(et_hbm.at[pl.ds(off, CH)], etb, sem2)
            c1.wait()
            c2.wait()
            g1 = pltpu.async_copy(inp_sh.at[esb], nib, sem1)
            g2 = pltpu.async_copy(inp_sh.at[etb], njb, sem2)
            g1.wait()
            g2.wait()
            w1 = pltpu.async_copy(nib, ni_hbm.at[pl.ds(off, CH)], sem3)
            w2 = pltpu.async_copy(njb, nj_hbm.at[pl.ds(off, CH)], sem4)
            w1.wait()
            w2.wait()
            return 0

        lax.fori_loop(0, NCHUNK, body, 0, unroll=False)

    return gather_pair


def _sc_gather_pair(inp, es, et):
    return _build_sc_gather_pair()(inp, es, et)


@functools.cache
def _build_sc_scatter(width):
    @functools.partial(
        pl.kernel, mesh=_mesh(),
        compiler_params=pltpu.CompilerParams(use_tc_tiling_on_sc=False),
        out_type=[jax.ShapeDtypeStruct((N, width), f32),
                  jax.ShapeDtypeStruct((N, width), f32)],
        scratch_types=[pltpu.VMEM((CH,), jnp.int32),
                       pltpu.VMEM((CH, width), f32),
                       pltpu.VMEM_SHARED((N, width), f32),
                       pltpu.SemaphoreType.DMA,
                       pltpu.SemaphoreType.DMA],
    )
    def scat(val_hbm, es_hbm, zero_hbm, p0_hbm, p1_hbm, idxb, vb, acc_sh,
             sem1, sem2):
        cid = lax.axis_index("c")
        sid = lax.axis_index("s")

        @pl.when(sid == 0)
        def _():
            pltpu.sync_copy(zero_hbm, acc_sh)

        plsc.subcore_barrier()
        base = _wid() * PER_W

        def body(j, _):
            off = base + j * CH
            c1 = pltpu.async_copy(es_hbm.at[pl.ds(off, CH)], idxb, sem1)
            c2 = pltpu.async_copy(val_hbm.at[pl.ds(off, CH)], vb, sem2)
            c1.wait()
            c2.wait()
            pltpu.sync_copy(vb, acc_sh.at[idxb], add=True)
            return 0

        lax.fori_loop(0, NCHUNK, body, 0, unroll=False)
        plsc.subcore_barrier()

        @pl.when((sid == 0) & (cid == 0))
        def _():
            pltpu.sync_copy(acc_sh, p0_hbm)

        @pl.when((sid == 0) & (cid == 1))
        def _():
            pltpu.sync_copy(acc_sh, p1_hbm)

    return scat


def _sc_scatter_h(val, es, zero):
    return _build_sc_scatter(HP)(val, es, zero)


def _sc_scatter_d(val, es, zero):
    return _build_sc_scatter(D)(val, es, zero)


@functools.cache
def _build_sc_gather_srow():
    @functools.partial(
        pl.kernel, mesh=_mesh(),
        compiler_params=pltpu.CompilerParams(use_tc_tiling_on_sc=False),
        out_type=[jax.ShapeDtypeStruct((E, HP), f32),
                  jax.ShapeDtypeStruct((E, HP), f32)],
        scratch_types=[pltpu.VMEM((CH,), jnp.int32),
                       pltpu.VMEM((CH, HP), f32),
                       pltpu.VMEM((CH, HP), f32),
                       pltpu.VMEM_SHARED((N, HP), f32),
                       pltpu.VMEM_SHARED((N, HP), f32),
                       pltpu.SemaphoreType.DMA,
                       pltpu.SemaphoreType.DMA,
                       pltpu.SemaphoreType.DMA,
                       pltpu.SemaphoreType.DMA],
    )
    def gather_srow(p0_hbm, p1_hbm, es_hbm, s0_hbm, s1_hbm,
                    idxb, b0, b1, p0_sh, p1_sh, sem1, sem2, sem3, sem4):
        sid = lax.axis_index("s")

        @pl.when(sid == 0)
        def _():
            pltpu.sync_copy(p0_hbm, p0_sh)

        @pl.when(sid == 1)
        def _():
            pltpu.sync_copy(p1_hbm, p1_sh)

        plsc.subcore_barrier()
        base = _wid() * PER_W

        def body(j, _):
            off = base + j * CH
            pltpu.sync_copy(es_hbm.at[pl.ds(off, CH)], idxb)
            g1 = pltpu.async_copy(p0_sh.at[idxb], b0, sem1)
            g2 = pltpu.async_copy(p1_sh.at[idxb], b1, sem2)
            g1.wait()
            g2.wait()
            w1 = pltpu.async_copy(b0, s0_hbm.at[pl.ds(off, CH)], sem3)
            w2 = pltpu.async_copy(b1, s1_hbm.at[pl.ds(off, CH)], sem4)
            w1.wait()
            w2.wait()
            return 0

        lax.fori_loop(0, NCHUNK, body, 0, unroll=False)

    return gather_srow


def _sc_gather_srow(p0, p1, es):
    return _build_sc_gather_srow()(p0, p1, es)


_INV_SQRT_BN = 1.0 / math.sqrt(1.0 + 1e-05)


def _sigmoid(x):
    return 1.0 / (1.0 + jnp.exp(-x))


def _softplus(x):
    return jnp.maximum(x, 0.0) + jnp.log(1.0 + jnp.exp(-jnp.abs(x)))


def _elu(x):
    return jnp.where(x > 0.0, x, jnp.exp(jnp.minimum(x, 0.0)) - 1.0)


def _edge_body(ni_ref, nj_ref, rij_ref, pw_ref, ea_ref,
               wg1_ref, wg2_ref, wg3_ref, wm1_ref, wm2_ref, wm3_ref,
               w2g_ref, w2_ref, wat_ref, wab_ref, a1_ref, a2_ref,
               e_ref, ap_ref, u_ref):
    ni = ni_ref[...]
    nj = nj_ref[...]
    r = rij_ref[...]
    r = r + (r == 0.0).astype(f32) * 1e-08
    delta = (ni - nj) / r
    pre_g = ni @ wg1_ref[...] + nj @ wg2_ref[...] + delta @ wg3_ref[...]
    pre_m = ni @ wm1_ref[...] + nj @ wm2_ref[...] + delta @ wm3_ref[...]
    gmat = _sigmoid(pre_g) * _elu(pre_m)
    pw = pw_ref[...]
    gate = _sigmoid(pw @ w2g_ref[...])
    z2 = (pw * gate) @ w2_ref[...]
    ea_w = ea_ref[...] @ wab_ref[...]
    oi = _softplus(ni @ wat_ref[...] + ea_w)
    oj = _softplus(nj @ wat_ref[...] + ea_w)
    alpha = oi @ a1_ref[...] + oj @ a2_ref[...]
    alpha = _softplus(_softplus(alpha) * _INV_SQRT_BN)
    e = jnp.exp(alpha)
    e_ref[:, :H] = e
    e_ref[:, H:] = jnp.zeros((e.shape[0], HP - H), f32)
    ap_ref[...] = gmat * z2
    for h in range(H):
        u_ref[:, h * D:(h + 1) * D] = (
            oj[:, h * D:(h + 1) * D] * gmat * ((1.0 / H) * e[:, h:h + 1]))


BE_A = 2560


def _tc_edge(ni, nj, rij2, pw, ea, wg1, wg2, wg3, wm1, wm2, wm3,
             w2g, w2, wat, wab, a1, a2):
    grid = (E // BE_A,)
    row = lambda i: (i, 0)
    full = lambda i: (0, 0)
    return pl.pallas_call(
        _edge_body,
        grid=grid,
        in_specs=[
            pl.BlockSpec((BE_A, D), row),
            pl.BlockSpec((BE_A, D), row),
            pl.BlockSpec((BE_A, 1), row),
            pl.BlockSpec((BE_A, D), row),
            pl.BlockSpec((BE_A, D), row),
            pl.BlockSpec((D, D), full), pl.BlockSpec((D, D), full),
            pl.BlockSpec((D, D), full), pl.BlockSpec((D, D), full),
            pl.BlockSpec((D, D), full), pl.BlockSpec((D, D), full),
            pl.BlockSpec((D, D), full), pl.BlockSpec((D, D), full),
            pl.BlockSpec((D, H * D), full), pl.BlockSpec((D, H * D), full),
            pl.BlockSpec((H * D, H), full), pl.BlockSpec((H * D, H), full),
        ],
        out_specs=[
            pl.BlockSpec((BE_A, HP), row),
            pl.BlockSpec((BE_A, D), row),
            pl.BlockSpec((BE_A, H * D), row),
        ],
        out_shape=[
            jax.ShapeDtypeStruct((E, HP), f32),
            jax.ShapeDtypeStruct((E, D), f32),
            jax.ShapeDtypeStruct((E, H * D), f32),
        ],
    )(ni, nj, rij2, pw, ea, wg1, wg2, wg3, wm1, wm2, wm3,
      w2g, w2, wat, wab, a1, a2)


def _combine_body(u_ref, ap_ref, s0_ref, s1_ref, z_ref):
    rcp = 1.0 / (s0_ref[:, :H] + s1_ref[:, :H] + 1e-16)
    z = ap_ref[...]
    u = u_ref[...]
    for h in range(H):
        z = z + u[:, h * D:(h + 1) * D] * rcp[:, h:h + 1]
    z_ref[...] = z


BE_B = 3200


def _tc_combine(u, ap, s0, s1):
    grid = (E // BE_B,)
    row = lambda i: (i, 0)
    return pl.pallas_call(
        _combine_body,
        grid=grid,
        in_specs=[
            pl.BlockSpec((BE_B, H * D), row),
            pl.BlockSpec((BE_B, D), row),
            pl.BlockSpec((BE_B, HP), row),
            pl.BlockSpec((BE_B, HP), row),
        ],
        out_specs=pl.BlockSpec((BE_B, D), row),
        out_shape=jax.ShapeDtypeStruct((E, D), f32),
    )(u, ap, s0, s1)


def _final_body(inp_ref, q0_ref, q1_ref, out_ref):
    out_ref[...] = inp_ref[...] + q0_ref[...] + q1_ref[...]


BN = 2000


def _tc_final(inp, q0, q1):
    grid = (N // BN,)
    row = lambda i: (i, 0)
    return pl.pallas_call(
        _final_body,
        grid=grid,
        in_specs=[pl.BlockSpec((BN, D), row)] * 3,
        out_specs=pl.BlockSpec((BN, D), row),
        out_shape=jax.ShapeDtypeStruct((N, D), f32),
    )(inp, q0, q1)


def kernel(input, nodes, edge_sources, edge_targets, rij, plane_wave,
           edge_attr, edge_index, Wg, Wm, W2g, W2, W_agat, att):
    es = edge_sources.astype(jnp.int32)
    et = edge_targets.astype(jnp.int32)
    rij2 = rij.reshape(E, 1)

    att2 = att[0]
    eye = jnp.eye(H, dtype=f32)
    a1 = jnp.einsum("hd,hg->hdg", att2[:, :D], eye).reshape(H * D, H)
    a2 = jnp.einsum("hd,hg->hdg", att2[:, D:], eye).reshape(H * D, H)

    wg1, wg2, wg3 = Wg[:D], Wg[D:2 * D], Wg[2 * D:]
    wm1, wm2, wm3 = Wm[:D], Wm[D:2 * D], Wm[2 * D:]
    wat, wab = W_agat[:D], W_agat[D:]

    ni, nj = _sc_gather_pair(input, es, et)
    e, ap, u = _tc_edge(ni, nj, rij2, plane_wave, edge_attr,
                        wg1, wg2, wg3, wm1, wm2, wm3, W2g, W2, wat, wab,
                        a1, a2)
    zero_h = jnp.zeros((N, HP), dtype=f32)
    p0, p1 = _sc_scatter_h(e, es, zero_h)
    s0, s1 = _sc_gather_srow(p0, p1, es)
    z = _tc_combine(u, ap, s0, s1)
    zero_d = jnp.zeros((N, D), dtype=f32)
    q0, q1 = _sc_scatter_d(z, es, zero_d)
    return _tc_final(input, q0, q1)


# speedup vs baseline: 3.2043x; 1.0761x over previous
import functools
import math

import jax
import jax.numpy as jnp
from jax import lax
from jax.experimental import pallas as pl
from jax.experimental.pallas import tpu as pltpu
from jax.experimental.pallas import tpu_sc as plsc

N = 10000
E = 320000
D = 64
H = 4

NC = 2
NS = 16
NW = NC * NS
PER_W = E // NW
RW = 125
RPT = PER_W // RW
HP = 16

f32 = jnp.float32


def _mesh():
    return plsc.VectorSubcoreMesh(core_axis_name="c", subcore_axis_name="s")


def _wid():
    return lax.axis_index("s") * NC + lax.axis_index("c")


@functools.cache
def _build_sc_gather_pair():
    RO = 4
    BLK = RO * RW

    @functools.partial(
        pl.kernel, mesh=_mesh(),
        compiler_params=pltpu.CompilerParams(use_tc_tiling_on_sc=False),
        out_type=[jax.ShapeDtypeStruct((E, D), f32),
                  jax.ShapeDtypeStruct((E, D), f32)],
        scratch_types=[pltpu.VMEM((RO, RW), jnp.int32),
                       pltpu.VMEM((RO, RW), jnp.int32),
                       pltpu.VMEM((BLK, D), f32),
                       pltpu.VMEM((BLK, D), f32),
                       pltpu.VMEM_SHARED((N, D), f32),
                       pltpu.SemaphoreType.DMA,
                       pltpu.SemaphoreType.DMA,
                       pltpu.SemaphoreType.DMA,
                       pltpu.SemaphoreType.DMA],
    )
    def gather_pair(inp_hbm, es_hbm, et_hbm, ni_hbm, nj_hbm,
                    esb, etb, nib, njb, inp_sh, sem1, sem2, sem3, sem4):
        sid = lax.axis_index("s")

        @pl.when(sid == 0)
        def _():
            pltpu.sync_copy(inp_hbm, inp_sh)

        plsc.subcore_barrier()
        wid = _wid()

        def body(o, _):
            r = wid * RPT + o * RO
            c1 = pltpu.async_copy(es_hbm.at[pl.ds(r, RO)], esb, sem1)
            c2 = pltpu.async_copy(et_hbm.at[pl.ds(r, RO)], etb, sem2)
            c1.wait()
            c2.wait()
            ds = []
            for k in range(RO):
                dst = pl.ds(k * RW, RW)
                ds.append(pltpu.async_copy(
                    inp_sh.at[esb.at[k]], nib.at[dst], sem3))
                ds.append(pltpu.async_copy(
                    inp_sh.at[etb.at[k]], njb.at[dst], sem4))
            for dcp in ds:
                dcp.wait()
            off = wid * PER_W + o * BLK
            w1 = pltpu.async_copy(nib, ni_hbm.at[pl.ds(off, BLK)], sem1)
            w2 = pltpu.async_copy(njb, nj_hbm.at[pl.ds(off, BLK)], sem2)
            w1.wait()
            w2.wait()
            return 0

        lax.fori_loop(0, RPT // RO, body, 0, unroll=False)

    return gather_pair


def _sc_gather_pair(inp, es, et):
    return _build_sc_gather_pair()(inp, es.reshape(E // RW, RW),
                                   et.reshape(E // RW, RW))


@functools.cache
def _build_sc_scatter(width):
    RO = 8
    BLK = RO * RW

    @functools.partial(
        pl.kernel, mesh=_mesh(),
        compiler_params=pltpu.CompilerParams(use_tc_tiling_on_sc=False),
        out_type=[jax.ShapeDtypeStruct((N, width), f32),
                  jax.ShapeDtypeStruct((N, width), f32)],
        scratch_types=[pltpu.VMEM((RO, RW), jnp.int32),
                       pltpu.VMEM((BLK, width), f32),
                       pltpu.VMEM_SHARED((N, width), f32),
                       pltpu.SemaphoreType.DMA,
                       pltpu.SemaphoreType.DMA],
    )
    def scat(val_hbm, es_hbm, zero_hbm, p0_hbm, p1_hbm, idxb, vb, acc_sh,
             sem1, sem2):
        cid = lax.axis_index("c")
        sid = lax.axis_index("s")

        @pl.when(sid == 0)
        def _():
            pltpu.sync_copy(zero_hbm, acc_sh)

        plsc.subcore_barrier()
        wid = _wid()

        def body(o, _):
            r = wid * RPT + o * RO
            off = wid * PER_W + o * BLK
            c1 = pltpu.async_copy(es_hbm.at[pl.ds(r, RO)], idxb, sem1)
            c2 = pltpu.async_copy(val_hbm.at[pl.ds(off, BLK)], vb, sem2)
            c1.wait()
            c2.wait()
            for k in range(RO):
                pltpu.sync_copy(vb.at[pl.ds(k * RW, RW)],
                                acc_sh.at[idxb.at[k]], add=True)
            return 0

        lax.fori_loop(0, RPT // RO, body, 0, unroll=False)
        plsc.subcore_barrier()

        @pl.when((sid == 0) & (cid == 0))
        def _():
            pltpu.sync_copy(acc_sh, p0_hbm)

        @pl.when((sid == 0) & (cid == 1))
        def _():
            pltpu.sync_copy(acc_sh, p1_hbm)

    return scat


def _sc_scatter_h(val, es, zero):
    return _build_sc_scatter(HP)(val, es.reshape(E // RW, RW), zero)


def _sc_scatter_d(val, es, zero):
    return _build_sc_scatter(D)(val, es.reshape(E // RW, RW), zero)


@functools.cache
def _build_sc_gather_srow():
    RO = 8
    BLK = RO * RW

    @functools.partial(
        pl.kernel, mesh=_mesh(),
        compiler_params=pltpu.CompilerParams(use_tc_tiling_on_sc=False),
        out_type=[jax.ShapeDtypeStruct((E, HP), f32),
                  jax.ShapeDtypeStruct((E, HP), f32)],
        scratch_types=[pltpu.VMEM((RO, RW), jnp.int32),
                       pltpu.VMEM((BLK, HP), f32),
                       pltpu.VMEM((BLK, HP), f32),
                       pltpu.VMEM_SHARED((N, HP), f32),
                       pltpu.VMEM_SHARED((N, HP), f32),
                       pltpu.SemaphoreType.DMA,
                       pltpu.SemaphoreType.DMA,
                       pltpu.SemaphoreType.DMA,
                       pltpu.SemaphoreType.DMA],
    )
    def gather_srow(p0_hbm, p1_hbm, es_hbm, s0_hbm, s1_hbm,
                    idxb, b0, b1, p0_sh, p1_sh, sem1, sem2, sem3, sem4):
        sid = lax.axis_index("s")

        @pl.when(sid == 0)
        def _():
            pltpu.sync_copy(p0_hbm, p0_sh)

        @pl.when(sid == 1)
        def _():
            pltpu.sync_copy(p1_hbm, p1_sh)

        plsc.subcore_barrier()
        wid = _wid()

        def body(o, _):
            r = wid * RPT + o * RO
            pltpu.sync_copy(es_hbm.at[pl.ds(r, RO)], idxb)
            ds = []
            for k in range(RO):
                dst = pl.ds(k * RW, RW)
                ds.append(pltpu.async_copy(
                    p0_sh.at[idxb.at[k]], b0.at[dst], sem1))
                ds.append(pltpu.async_copy(
                    p1_sh.at[idxb.at[k]], b1.at[dst], sem2))
            for dcp in ds:
                dcp.wait()
            off = wid * PER_W + o * BLK
            w1 = pltpu.async_copy(b0, s0_hbm.at[pl.ds(off, BLK)], sem3)
            w2 = pltpu.async_copy(b1, s1_hbm.at[pl.ds(off, BLK)], sem4)
            w1.wait()
            w2.wait()
            return 0

        lax.fori_loop(0, RPT // RO, body, 0, unroll=False)

    return gather_srow


def _sc_gather_srow(p0, p1, es):
    return _build_sc_gather_srow()(p0, p1, es.reshape(E // RW, RW))


_INV_SQRT_BN = 1.0 / math.sqrt(1.0 + 1e-05)


def _sigmoid(x):
    return 1.0 / (1.0 + jnp.exp(-x))


def _softplus(x):
    return jnp.maximum(x, 0.0) + jnp.log(1.0 + jnp.exp(-jnp.abs(x)))


def _elu(x):
    return jnp.where(x > 0.0, x, jnp.exp(jnp.minimum(x, 0.0)) - 1.0)


def _edge_body(ni_ref, nj_ref, rij_ref, pw_ref, ea_ref,
               wg1_ref, wg2_ref, wg3_ref, wm1_ref, wm2_ref, wm3_ref,
               w2g_ref, w2_ref, wat_ref, wab_ref, a1_ref, a2_ref,
               e_ref, ap_ref, u_ref):
    ni = ni_ref[...]
    nj = nj_ref[...]
    r = rij_ref[...]
    r = r + (r == 0.0).astype(f32) * 1e-08
    delta = (ni - nj) / r
    pre_g = ni @ wg1_ref[...] + nj @ wg2_ref[...] + delta @ wg3_ref[...]
    pre_m = ni @ wm1_ref[...] + nj @ wm2_ref[...] + delta @ wm3_ref[...]
    gmat = _sigmoid(pre_g) * _elu(pre_m)
    pw = pw_ref[...]
    gate = _sigmoid(pw @ w2g_ref[...])
    z2 = (pw * gate) @ w2_ref[...]
    ea_w = ea_ref[...] @ wab_ref[...]
    oi = _softplus(ni @ wat_ref[...] + ea_w)
    oj = _softplus(nj @ wat_ref[...] + ea_w)
    alpha = oi @ a1_ref[...] + oj @ a2_ref[...]
    alpha = _softplus(_softplus(alpha) * _INV_SQRT_BN)
    e = jnp.exp(alpha)
    e_ref[:, :H] = e
    e_ref[:, H:] = jnp.zeros((e.shape[0], HP - H), f32)
    ap_ref[...] = gmat * z2
    for h in range(H):
        u_ref[:, h * D:(h + 1) * D] = (
            oj[:, h * D:(h + 1) * D] * gmat * ((1.0 / H) * e[:, h:h + 1]))


BE_A = 2560


def _tc_edge(ni, nj, rij2, pw, ea, wg1, wg2, wg3, wm1, wm2, wm3,
             w2g, w2, wat, wab, a1, a2):
    grid = (E // BE_A,)
    row = lambda i: (i, 0)
    full = lambda i: (0, 0)
    return pl.pallas_call(
        _edge_body,
        grid=grid,
        in_specs=[
            pl.BlockSpec((BE_A, D), row),
            pl.BlockSpec((BE_A, D), row),
            pl.BlockSpec((BE_A, 1), row),
            pl.BlockSpec((BE_A, D), row),
            pl.BlockSpec((BE_A, D), row),
            pl.BlockSpec((D, D), full), pl.BlockSpec((D, D), full),
            pl.BlockSpec((D, D), full), pl.BlockSpec((D, D), full),
            pl.BlockSpec((D, D), full), pl.BlockSpec((D, D), full),
            pl.BlockSpec((D, D), full), pl.BlockSpec((D, D), full),
            pl.BlockSpec((D, H * D), full), pl.BlockSpec((D, H * D), full),
            pl.BlockSpec((H * D, H), full), pl.BlockSpec((H * D, H), full),
        ],
        out_specs=[
            pl.BlockSpec((BE_A, HP), row),
            pl.BlockSpec((BE_A, D), row),
            pl.BlockSpec((BE_A, H * D), row),
        ],
        out_shape=[
            jax.ShapeDtypeStruct((E, HP), f32),
            jax.ShapeDtypeStruct((E, D), f32),
            jax.ShapeDtypeStruct((E, H * D), f32),
        ],
    )(ni, nj, rij2, pw, ea, wg1, wg2, wg3, wm1, wm2, wm3,
      w2g, w2, wat, wab, a1, a2)


def _combine_body(u_ref, ap_ref, s0_ref, s1_ref, z_ref):
    rcp = 1.0 / (s0_ref[:, :H] + s1_ref[:, :H] + 1e-16)
    z = ap_ref[...]
    u = u_ref[...]
    for h in range(H):
        z = z + u[:, h * D:(h + 1) * D] * rcp[:, h:h + 1]
    z_ref[...] = z


BE_B = 3200


def _tc_combine(u, ap, s0, s1):
    grid = (E // BE_B,)
    row = lambda i: (i, 0)
    return pl.pallas_call(
        _combine_body,
        grid=grid,
        in_specs=[
            pl.BlockSpec((BE_B, H * D), row),
            pl.BlockSpec((BE_B, D), row),
            pl.BlockSpec((BE_B, HP), row),
            pl.BlockSpec((BE_B, HP), row),
        ],
        out_specs=pl.BlockSpec((BE_B, D), row),
        out_shape=jax.ShapeDtypeStruct((E, D), f32),
    )(u, ap, s0, s1)


def _final_body(inp_ref, q0_ref, q1_ref, out_ref):
    out_ref[...] = inp_ref[...] + q0_ref[...] + q1_ref[...]


BN = 2000


def _tc_final(inp, q0, q1):
    grid = (N // BN,)
    row = lambda i: (i, 0)
    return pl.pallas_call(
        _final_body,
        grid=grid,
        in_specs=[pl.BlockSpec((BN, D), row)] * 3,
        out_specs=pl.BlockSpec((BN, D), row),
        out_shape=jax.ShapeDtypeStruct((N, D), f32),
    )(inp, q0, q1)


def kernel(input, nodes, edge_sources, edge_targets, rij, plane_wave,
           edge_attr, edge_index, Wg, Wm, W2g, W2, W_agat, att):
    es = edge_sources.astype(jnp.int32)
    et = edge_targets.astype(jnp.int32)
    rij2 = rij.reshape(E, 1)

    att2 = att[0]
    eye = jnp.eye(H, dtype=f32)
    a1 = jnp.einsum("hd,hg->hdg", att2[:, :D], eye).reshape(H * D, H)
    a2 = jnp.einsum("hd,hg->hdg", att2[:, D:], eye).reshape(H * D, H)

    wg1, wg2, wg3 = Wg[:D], Wg[D:2 * D], Wg[2 * D:]
    wm1, wm2, wm3 = Wm[:D], Wm[D:2 * D], Wm[2 * D:]
    wat, wab = W_agat[:D], W_agat[D:]

    ni, nj = _sc_gather_pair(input, es, et)
    e, ap, u = _tc_edge(ni, nj, rij2, plane_wave, edge_attr,
                        wg1, wg2, wg3, wm1, wm2, wm3, W2g, W2, wat, wab,
                        a1, a2)
    zero_h = jnp.zeros((N, HP), dtype=f32)
    p0, p1 = _sc_scatter_h(e, es, zero_h)
    s0, s1 = _sc_gather_srow(p0, p1, es)
    z = _tc_combine(u, ap, s0, s1)
    zero_d = jnp.zeros((N, D), dtype=f32)
    q0, q1 = _sc_scatter_d(z, es, zero_d)
    return _tc_final(input, q0, q1)


# speedup vs baseline: 3.2298x; 1.0080x over previous
import functools
import math

import jax
import jax.numpy as jnp
from jax import lax
from jax.experimental import pallas as pl
from jax.experimental.pallas import tpu as pltpu
from jax.experimental.pallas import tpu_sc as plsc

N = 10000
E = 320000
D = 64
H = 4

NC = 2
NS = 16
NW = NC * NS
PER_W = E // NW
RW = 125
RPT = PER_W // RW
HP = 16

f32 = jnp.float32


def _mesh():
    return plsc.VectorSubcoreMesh(core_axis_name="c", subcore_axis_name="s")


def _wid():
    return lax.axis_index("s") * NC + lax.axis_index("c")


@functools.cache
def _build_sc_gather_pair():
    RO = 4
    BLK = RO * RW

    @functools.partial(
        pl.kernel, mesh=_mesh(),
        compiler_params=pltpu.CompilerParams(use_tc_tiling_on_sc=False),
        out_type=[jax.ShapeDtypeStruct((E, D), f32),
                  jax.ShapeDtypeStruct((E, D), f32)],
        scratch_types=[pltpu.VMEM((RO, RW), jnp.int32),
                       pltpu.VMEM((RO, RW), jnp.int32),
                       pltpu.VMEM((BLK, D), f32),
                       pltpu.VMEM((BLK, D), f32),
                       pltpu.VMEM_SHARED((N, D), f32),
                       pltpu.SemaphoreType.DMA,
                       pltpu.SemaphoreType.DMA,
                       pltpu.SemaphoreType.DMA,
                       pltpu.SemaphoreType.DMA],
    )
    def gather_pair(inp_hbm, es_hbm, et_hbm, ni_hbm, nj_hbm,
                    esb, etb, nib, njb, inp_sh, sem1, sem2, sem3, sem4):
        sid = lax.axis_index("s")

        @pl.when(sid == 0)
        def _():
            pltpu.sync_copy(inp_hbm, inp_sh)

        plsc.subcore_barrier()
        wid = _wid()

        def body(o, _):
            r = wid * RPT + o * RO
            c1 = pltpu.async_copy(es_hbm.at[pl.ds(r, RO)], esb, sem1)
            c2 = pltpu.async_copy(et_hbm.at[pl.ds(r, RO)], etb, sem2)
            c1.wait()
            c2.wait()
            ds = []
            for k in range(RO):
                dst = pl.ds(k * RW, RW)
                ds.append(pltpu.async_copy(
                    inp_sh.at[esb.at[k]], nib.at[dst], sem3))
                ds.append(pltpu.async_copy(
                    inp_sh.at[etb.at[k]], njb.at[dst], sem4))
            for dcp in ds:
                dcp.wait()
            off = wid * PER_W + o * BLK
            w1 = pltpu.async_copy(nib, ni_hbm.at[pl.ds(off, BLK)], sem1)
            w2 = pltpu.async_copy(njb, nj_hbm.at[pl.ds(off, BLK)], sem2)
            w1.wait()
            w2.wait()
            return 0

        lax.fori_loop(0, RPT // RO, body, 0, unroll=False)

    return gather_pair


def _sc_gather_pair(inp, es, et):
    return _build_sc_gather_pair()(inp, es.reshape(E // RW, RW),
                                   et.reshape(E // RW, RW))


@functools.cache
def _build_sc_scatter(width):
    RO = 8
    BLK = RO * RW

    @functools.partial(
        pl.kernel, mesh=_mesh(),
        compiler_params=pltpu.CompilerParams(use_tc_tiling_on_sc=False),
        out_type=[jax.ShapeDtypeStruct((N, width), f32),
                  jax.ShapeDtypeStruct((N, width), f32)],
        scratch_types=[pltpu.VMEM((RO, RW), jnp.int32),
                       pltpu.VMEM((BLK, width), f32),
                       pltpu.VMEM_SHARED((N, width), f32),
                       pltpu.SemaphoreType.DMA,
                       pltpu.SemaphoreType.DMA],
    )
    def scat(val_hbm, es_hbm, zero_hbm, p0_hbm, p1_hbm, idxb, vb, acc_sh,
             sem1, sem2):
        cid = lax.axis_index("c")
        sid = lax.axis_index("s")

        @pl.when(sid == 0)
        def _():
            pltpu.sync_copy(zero_hbm, acc_sh)

        plsc.subcore_barrier()
        wid = _wid()

        def body(o, _):
            r = wid * RPT + o * RO
            off = wid * PER_W + o * BLK
            c1 = pltpu.async_copy(es_hbm.at[pl.ds(r, RO)], idxb, sem1)
            c2 = pltpu.async_copy(val_hbm.at[pl.ds(off, BLK)], vb, sem2)
            c1.wait()
            c2.wait()
            for k in range(RO):
                pltpu.sync_copy(vb.at[pl.ds(k * RW, RW)],
                                acc_sh.at[idxb.at[k]], add=True)
            return 0

        lax.fori_loop(0, RPT // RO, body, 0, unroll=False)
        plsc.subcore_barrier()

        @pl.when((sid == 0) & (cid == 0))
        def _():
            pltpu.sync_copy(acc_sh, p0_hbm)

        @pl.when((sid == 0) & (cid == 1))
        def _():
            pltpu.sync_copy(acc_sh, p1_hbm)

    return scat


def _sc_scatter_h(val, es, zero):
    return _build_sc_scatter(HP)(val, es.reshape(E // RW, RW), zero)


def _sc_scatter_d(val, es, zero):
    return _build_sc_scatter(D)(val, es.reshape(E // RW, RW), zero)


@functools.cache
def _build_sc_gather_srow():
    RO = 8
    BLK = RO * RW

    @functools.partial(
        pl.kernel, mesh=_mesh(),
        compiler_params=pltpu.CompilerParams(use_tc_tiling_on_sc=False),
        out_type=[jax.ShapeDtypeStruct((E, HP), f32),
                  jax.ShapeDtypeStruct((E, HP), f32)],
        scratch_types=[pltpu.VMEM((RO, RW), jnp.int32),
                       pltpu.VMEM((BLK, HP), f32),
                       pltpu.VMEM((BLK, HP), f32),
                       pltpu.VMEM_SHARED((N, HP), f32),
                       pltpu.VMEM_SHARED((N, HP), f32),
                       pltpu.SemaphoreType.DMA,
                       pltpu.SemaphoreType.DMA,
                       pltpu.SemaphoreType.DMA,
                       pltpu.SemaphoreType.DMA],
    )
    def gather_srow(p0_hbm, p1_hbm, es_hbm, s0_hbm, s1_hbm,
                    idxb, b0, b1, p0_sh, p1_sh, sem1, sem2, sem3, sem4):
        sid = lax.axis_index("s")

        @pl.when(sid == 0)
        def _():
            pltpu.sync_copy(p0_hbm, p0_sh)

        @pl.when(sid == 1)
        def _():
            pltpu.sync_copy(p1_hbm, p1_sh)

        plsc.subcore_barrier()
        wid = _wid()

        def body(o, _):
            r = wid * RPT + o * RO
            pltpu.sync_copy(es_hbm.at[pl.ds(r, RO)], idxb)
            ds = []
            for k in range(RO):
                dst = pl.ds(k * RW, RW)
                ds.append(pltpu.async_copy(
                    p0_sh.at[idxb.at[k]], b0.at[dst], sem1))
                ds.append(pltpu.async_copy(
                    p1_sh.at[idxb.at[k]], b1.at[dst], sem2))
            for dcp in ds:
                dcp.wait()
            off = wid * PER_W + o * BLK
            w1 = pltpu.async_copy(b0, s0_hbm.at[pl.ds(off, BLK)], sem3)
            w2 = pltpu.async_copy(b1, s1_hbm.at[pl.ds(off, BLK)], sem4)
            w1.wait()
            w2.wait()
            return 0

        lax.fori_loop(0, RPT // RO, body, 0, unroll=False)

    return gather_srow


def _sc_gather_srow(p0, p1, es):
    return _build_sc_gather_srow()(p0, p1, es.reshape(E // RW, RW))


_INV_SQRT_BN = 1.0 / math.sqrt(1.0 + 1e-05)


def _sigmoid(x):
    return 1.0 / (1.0 + jnp.exp(-x))


def _softplus(x):
    return jnp.maximum(x, 0.0) + jnp.log(1.0 + jnp.exp(-jnp.abs(x)))


def _elu(x):
    return jnp.where(x > 0.0, x, jnp.exp(jnp.minimum(x, 0.0)) - 1.0)


def _edge_body(ni_ref, nj_ref, rij_ref, pw_ref, ea_ref,
               wg1_ref, wg2_ref, wg3_ref, wm1_ref, wm2_ref, wm3_ref,
               w2g_ref, w2_ref, wat_ref, wab_ref, a1_ref, a2_ref,
               e_ref, ap_ref, u_ref):
    ni = ni_ref[...]
    nj = nj_ref[...]
    r = rij_ref[...]
    r = r + (r == 0.0).astype(f32) * 1e-08
    delta = (ni - nj) / r
    pre_g = ni @ wg1_ref[...] + nj @ wg2_ref[...] + delta @ wg3_ref[...]
    pre_m = ni @ wm1_ref[...] + nj @ wm2_ref[...] + delta @ wm3_ref[...]
    gmat = _sigmoid(pre_g) * _elu(pre_m)
    pw = pw_ref[...]
    gate = _sigmoid(pw @ w2g_ref[...])
    z2 = (pw * gate) @ w2_ref[...]
    ea_w = ea_ref[...] @ wab_ref[...]
    oi = _softplus(ni @ wat_ref[...] + ea_w)
    oj = _softplus(nj @ wat_ref[...] + ea_w)
    alpha = oi @ a1_ref[...] + oj @ a2_ref[...]
    alpha = _softplus(_softplus(alpha) * _INV_SQRT_BN)
    e = jnp.exp(alpha)
    e_ref[:, :H] = e
    e_ref[:, H:] = jnp.zeros((e.shape[0], HP - H), f32)
    ap_ref[...] = gmat * z2
    for h in range(H):
        u_ref[:, h * D:(h + 1) * D] = (
            oj[:, h * D:(h + 1) * D] * gmat
            * ((1.0 / H) * e[:, h:h + 1])).astype(jnp.bfloat16)


BE_A = 2560


def _tc_edge(ni, nj, rij2, pw, ea, wg1, wg2, wg3, wm1, wm2, wm3,
             w2g, w2, wat, wab, a1, a2):
    grid = (E // BE_A,)
    row = lambda i: (i, 0)
    full = lambda i: (0, 0)
    return pl.pallas_call(
        _edge_body,
        grid=grid,
        in_specs=[
            pl.BlockSpec((BE_A, D), row),
            pl.BlockSpec((BE_A, D), row),
            pl.BlockSpec((BE_A, 1), row),
            pl.BlockSpec((BE_A, D), row),
            pl.BlockSpec((BE_A, D), row),
            pl.BlockSpec((D, D), full), pl.BlockSpec((D, D), full),
            pl.BlockSpec((D, D), full), pl.BlockSpec((D, D), full),
            pl.BlockSpec((D, D), full), pl.BlockSpec((D, D), full),
            pl.BlockSpec((D, D), full), pl.BlockSpec((D, D), full),
            pl.BlockSpec((D, H * D), full), pl.BlockSpec((D, H * D), full),
            pl.BlockSpec((H * D, H), full), pl.BlockSpec((H * D, H), full),
        ],
        out_specs=[
            pl.BlockSpec((BE_A, HP), row),
            pl.BlockSpec((BE_A, D), row),
            pl.BlockSpec((BE_A, H * D), row),
        ],
        out_shape=[
            jax.ShapeDtypeStruct((E, HP), f32),
            jax.ShapeDtypeStruct((E, D), f32),
            jax.ShapeDtypeStruct((E, H * D), jnp.bfloat16),
        ],
    )(ni, nj, rij2, pw, ea, wg1, wg2, wg3, wm1, wm2, wm3,
      w2g, w2, wat, wab, a1, a2)


def _combine_body(u_ref, ap_ref, s0_ref, s1_ref, z_ref):
    rcp = 1.0 / (s0_ref[:, :H] + s1_ref[:, :H] + 1e-16)
    z = ap_ref[...]
    u = u_ref[...].astype(f32)
    for h in range(H):
        z = z + u[:, h * D:(h + 1) * D] * rcp[:, h:h + 1]
    z_ref[...] = z


BE_B = 3200


def _tc_combine(u, ap, s0, s1):
    grid = (E // BE_B,)
    row = lambda i: (i, 0)
    return pl.pallas_call(
        _combine_body,
        grid=grid,
        in_specs=[
            pl.BlockSpec((BE_B, H * D), row),
            pl.BlockSpec((BE_B, D), row),
            pl.BlockSpec((BE_B, HP), row),
            pl.BlockSpec((BE_B, HP), row),
        ],
        out_specs=pl.BlockSpec((BE_B, D), row),
        out_shape=jax.ShapeDtypeStruct((E, D), f32),
    )(u, ap, s0, s1)


def _final_body(inp_ref, q0_ref, q1_ref, out_ref):
    out_ref[...] = inp_ref[...] + q0_ref[...] + q1_ref[...]


BN = 2000


def _tc_final(inp, q0, q1):
    grid = (N // BN,)
    row = lambda i: (i, 0)
    return pl.pallas_call(
        _final_body,
        grid=grid,
        in_specs=[pl.BlockSpec((BN, D), row)] * 3,
        out_specs=pl.BlockSpec((BN, D), row),
        out_shape=jax.ShapeDtypeStruct((N, D), f32),
    )(inp, q0, q1)


def kernel(input, nodes, edge_sources, edge_targets, rij, plane_wave,
           edge_attr, edge_index, Wg, Wm, W2g, W2, W_agat, att):
    es = edge_sources.astype(jnp.int32)
    et = edge_targets.astype(jnp.int32)
    rij2 = rij.reshape(E, 1)

    att2 = att[0]
    eye = jnp.eye(H, dtype=f32)
    a1 = jnp.einsum("hd,hg->hdg", att2[:, :D], eye).reshape(H * D, H)
    a2 = jnp.einsum("hd,hg->hdg", att2[:, D:], eye).reshape(H * D, H)

    wg1, wg2, wg3 = Wg[:D], Wg[D:2 * D], Wg[2 * D:]
    wm1, wm2, wm3 = Wm[:D], Wm[D:2 * D], Wm[2 * D:]
    wat, wab = W_agat[:D], W_agat[D:]

    ni, nj = _sc_gather_pair(input, es, et)
    e, ap, u = _tc_edge(ni, nj, rij2, plane_wave, edge_attr,
                        wg1, wg2, wg3, wm1, wm2, wm3, W2g, W2, wat, wab,
                        a1, a2)
    zero_h = jnp.zeros((N, HP), dtype=f32)
    p0, p1 = _sc_scatter_h(e, es, zero_h)
    s0, s1 = _sc_gather_srow(p0, p1, es)
    z = _tc_combine(u, ap, s0, s1)
    zero_d = jnp.zeros((N, D), dtype=f32)
    q0, q1 = _sc_scatter_d(z, es, zero_d)
    return _tc_final(input, q0, q1)
